# baseline (speedup 1.0000x reference)
import functools

import jax
import jax.numpy as jnp
from jax.experimental import pallas as pl
from jax.experimental.pallas import tpu as pltpu

_B, _V, _S, _D = 64, 100000, 32, 768
_R = 8
_IMIN = -2147483648
_KEY_NEG_INF = -2139095040
_KEY_POS_INF = 2139095040
_TOPK = 50
_P = 0.9


def _order_key(x):
    b = jax.lax.bitcast_convert_type(x, jnp.int32)
    return jnp.where(b >= 0, b, jnp.int32(_IMIN) - b)


def _midpoint(lo, hi):
    return (lo >> 1) + (hi >> 1) + (lo & hi & 1)


def _body(t_ref, l_ref, g0_ref, g1_ref, g2_ref, g3_ref,
          out_ref, e_ref, key_ref):
    t = t_ref[0, 0]
    l = l_ref[...] / t
    key = _order_key(l)
    key_ref[...] = key
    m = jnp.max(l, axis=-1, keepdims=True)
    e = jnp.exp(l - m)
    e_ref[...] = e
    z = jnp.sum(e, axis=-1, keepdims=True)
    pz = jnp.float32(_P) * z

    ones = jnp.ones((_R, 1), dtype=jnp.int32)
    lo0 = ones * _KEY_NEG_INF
    hi0 = ones * _KEY_POS_INF

    def it(_, carry):
        lo_k, hi_k, lo_n, hi_n = carry
        kk = key_ref[...]
        ee = e_ref[...]

        k2 = _midpoint(lo_k, hi_k)
        k1 = _midpoint(lo_k, k2)
        k3 = _midpoint(k2, hi_k)
        one, zero = jnp.float32(1.0), jnp.float32(0.0)
        c1 = jnp.sum(jnp.where(kk > k1, one, zero), axis=-1, keepdims=True)
        c2 = jnp.sum(jnp.where(kk > k2, one, zero), axis=-1, keepdims=True)
        c3 = jnp.sum(jnp.where(kk > k3, one, zero), axis=-1, keepdims=True)
        kt = jnp.float32(_TOPK)
        b1, b2, b3 = c1 >= kt, c2 >= kt, c3 >= kt
        lo_k = jnp.where(b3, k3, jnp.where(b2, k2, jnp.where(b1, k1, lo_k)))
        hi_k = jnp.where(~b1, k1, jnp.where(~b2, k2, jnp.where(~b3, k3, hi_k)))

        n2 = _midpoint(lo_n, hi_n)
        n1 = _midpoint(lo_n, n2)
        n3 = _midpoint(n2, hi_n)
        g1 = jnp.sum(jnp.where(kk > n1, ee, zero), axis=-1, keepdims=True)
        g2 = jnp.sum(jnp.where(kk > n2, ee, zero), axis=-1, keepdims=True)
        g3 = jnp.sum(jnp.where(kk > n3, ee, zero), axis=-1, keepdims=True)
        d1, d2, d3 = g1 > pz, g2 > pz, g3 > pz
        lo_n = jnp.where(d3, n3, jnp.where(d2, n2, jnp.where(d1, n1, lo_n)))
        hi_n = jnp.where(~d1, n1, jnp.where(~d2, n2, jnp.where(~d3, n3, hi_n)))
        return lo_k, hi_k, lo_n, hi_n

    lo_k, _, lo_n, _ = jax.lax.fori_loop(0, 17, it, (lo0, hi0, lo0, hi0))

    keep_k = key > lo_k
    keep_n = key > lo_n

    probs = e / z
    logp = jnp.log(probs + jnp.float32(1e-10))
    ent = -jnp.sum(probs * logp, axis=-1, keepdims=True)
    keep_y = jnp.abs(-logp - ent) < jnp.float32(0.5)

    neg_inf = jnp.float32(-jnp.inf)
    iota = jax.lax.broadcasted_iota(jnp.int32, (_R, _V), 1)
    sentinel = jnp.int32(_V)

    def sample(keep, g_ref):
        g = -jnp.log(-jnp.log(g_ref[...]))
        vals = jnp.where(keep, l, neg_inf) + g
        mx = jnp.max(vals, axis=-1, keepdims=True)
        return jnp.min(jnp.where(vals == mx, iota, sentinel),
                       axis=-1, keepdims=True)

    s_n = sample(keep_n, g0_ref)
    s_k = sample(keep_k, g1_ref)
    s_t = sample(jnp.ones((_R, _V), dtype=jnp.bool_), g2_ref)
    s_y = sample(keep_y, g3_ref)
    out_ref[...] = jnp.concatenate([s_n, s_k, s_t, s_y], axis=-1)


def _mlp_body(h_ref, w1_ref, b1_ref, w2_ref, b2_ref, s_ref, out_ref):
    h = jnp.mean(h_ref[...], axis=1)
    z1 = jax.nn.relu(
        jnp.dot(h, w1_ref[...], preferred_element_type=jnp.float32)
        + b1_ref[...])
    z2 = (jnp.dot(z1, w2_ref[...], preferred_element_type=jnp.float32)
          + b2_ref[...])
    w = jax.nn.softmax(z2, axis=-1)
    samples = s_ref[...].astype(jnp.float32)
    weighted = jnp.sum(samples * w, axis=-1, keepdims=True)
    out_ref[...] = weighted.astype(jnp.int32)


@functools.partial(jax.jit, static_argnames=())
def kernel(logits, hidden_states, W1, b1, W2, b2, temperature=1.0):
    skey = jax.random.key(42)
    tiny = jnp.finfo(jnp.float32).tiny
    g = [jax.random.uniform(jax.random.fold_in(skey, i), (_B, _V), jnp.float32,
                            minval=tiny, maxval=1.0)
         for i in range(4)]
    t = jnp.asarray(temperature, jnp.float32).reshape(1, 1)
    b1r = b1.reshape(1, 256)
    b2r = b2.reshape(1, 4)

    grid = _B // _R
    row_spec = pl.BlockSpec((_R, _V), lambda i: (i, 0))
    samples = pl.pallas_call(
        _body,
        grid=(grid,),
        in_specs=[
            pl.BlockSpec((1, 1), lambda i: (0, 0)),
            row_spec, row_spec, row_spec, row_spec, row_spec,
        ],
        out_specs=pl.BlockSpec((_R, 4), lambda i: (i, 0)),
        out_shape=jax.ShapeDtypeStruct((_B, 4), jnp.int32),
        scratch_shapes=[
            pltpu.VMEM((_R, _V), jnp.float32),
            pltpu.VMEM((_R, _V), jnp.int32),
        ],
    )(t, logits, g[0], g[1], g[2], g[3])

    out = pl.pallas_call(
        _mlp_body,
        in_specs=[
            pl.BlockSpec((_B, _S, _D), lambda: (0, 0, 0)),
            pl.BlockSpec((_D, 256), lambda: (0, 0)),
            pl.BlockSpec((1, 256), lambda: (0, 0)),
            pl.BlockSpec((256, 4), lambda: (0, 0)),
            pl.BlockSpec((1, 4), lambda: (0, 0)),
            pl.BlockSpec((_B, 4), lambda: (0, 0)),
        ],
        out_specs=pl.BlockSpec((_B, 1), lambda: (0, 0)),
        out_shape=jax.ShapeDtypeStruct((_B, 1), jnp.int32),
    )(hidden_states, W1, b1r, W2, b2r, samples)
    return out.reshape(_B)
